# Initial kernel scaffold; baseline (speedup 1.0000x reference)
import jax
import jax.numpy as jnp
from jax.experimental import pallas as pl


def kernel(x, edge_index, edge_values, W_map, a1, b1, a2, b2, kernel, bias):
    raise NotImplementedError("write your pallas kernel here")


# speedup vs baseline: 23.0548x; 23.0548x over previous
import jax
import jax.numpy as jnp
from jax import lax
from jax.experimental import pallas as pl
from jax.experimental.pallas import tpu as pltpu
from jax.experimental.pallas import tpu_sc as plsc

N = 10000
E = 320000
D = 128
NC = 2
NS = 16
NW = NC * NS
L = 16
CH = 128
NCH = 79
EPT = NCH * CH
EPAD = NW * EPT
NPAD = 10240
ROWS_PT = NPAD // NS

GP = 10
NBLK = N // GP
EB = E // GP
GC = 10
CB = NPAD // GC


def _prep_body(x_ref, wmap_ref, a12_ref, kern_ref, ev_ref,
               val_ref, sa_ref, mx_ref):
    xb = x_ref[...]
    w12 = jnp.dot(wmap_ref[...], a12_ref[...],
                  preferred_element_type=jnp.float32)
    sab = jnp.dot(xb, w12, preferred_element_type=jnp.float32)
    sa_ref[...] = sab
    val_ref[...] = jnp.dot(xb, kern_ref[...],
                           preferred_element_type=jnp.float32)
    m1 = jnp.max(jnp.abs(sab[:, 0]))
    m2 = jnp.max(jnp.abs(sab[:, 1]))
    mev = jnp.max(jnp.abs(ev_ref[...]))
    z = jnp.float32(0.0)
    row = jnp.stack([m1, m2, mev, z, z, z, z, z])[None, :]
    mx_ref[pl.ds(pl.program_id(0), 1), :] = row


_prep = pl.pallas_call(
    _prep_body,
    grid=(GP,),
    in_specs=[
        pl.BlockSpec((NBLK, D), lambda i: (i, 0)),
        pl.BlockSpec((D, D), lambda i: (0, 0)),
        pl.BlockSpec((D, 2), lambda i: (0, 0)),
        pl.BlockSpec((D, D), lambda i: (0, 0)),
        pl.BlockSpec((1, 8, EB // 8), lambda i: (i, 0, 0)),
    ],
    out_specs=[
        pl.BlockSpec((NBLK, D), lambda i: (i, 0)),
        pl.BlockSpec((NBLK, 2), lambda i: (i, 0)),
        pl.BlockSpec((GP, 8), lambda i: (0, 0)),
    ],
    out_shape=[
        jax.ShapeDtypeStruct((N, D), jnp.float32),
        jax.ShapeDtypeStruct((N, 2), jnp.float32),
        jax.ShapeDtypeStruct((GP, 8), jnp.float32),
    ],
)


def _sc_body(src_ref, dst_ref, ev_ref, sa1_ref, sa2_ref, val_ref, bsh_ref,
             u_ref, d_ref,
             sa1_v, sa2_v, bsv, srcv, dstv, evv, exv, rows, spU, spd, gsem):
    cid = lax.axis_index("c")
    sid = lax.axis_index("s")
    wid = cid * NS + sid

    zeros16 = jnp.zeros((L,), jnp.float32)

    def _zero_row(r, carry):
        for c8 in range(D // L):
            rows[r, pl.ds(c8 * L, L)] = zeros16
        return carry

    lax.fori_loop(0, CH, _zero_row, 0)
    for c8 in range(CH // L):
        exv[pl.ds(c8 * L, L)] = zeros16

    for k in range(ROWS_PT // CH):
        pltpu.sync_copy(rows, spU.at[pl.ds(sid * ROWS_PT + k * CH, CH)])
        pltpu.sync_copy(exv, spd.at[pl.ds(sid * ROWS_PT + k * CH, CH)])

    pltpu.sync_copy(sa1_ref, sa1_v)
    pltpu.sync_copy(sa2_ref, sa2_v)
    pltpu.sync_copy(bsh_ref, bsv)
    plsc.subcore_barrier()

    lanes = jnp.arange(L, dtype=jnp.int32)

    def _chunk(c, carry):
        pltpu.sync_copy(src_ref.at[wid, c], srcv)
        pltpu.sync_copy(dst_ref.at[wid, c], dstv)
        pltpu.sync_copy(ev_ref.at[wid, c], evv)
        cp = pltpu.async_copy(val_ref.at[dstv], rows, gsem)
        bs = bsv[...]
        base = wid * EPT + c * CH
        for i in range(CH // L):
            s16 = srcv[pl.ds(i * L, L)]
            d16 = dstv[pl.ds(i * L, L)]
            e16 = evv[pl.ds(i * L, L)]
            g1 = plsc.load_gather(sa1_v, [s16])
            g2 = plsc.load_gather(sa2_v, [d16])
            e = e16 * g1 + e16 * g2
            lg = jnp.maximum(e, 0.0) + 0.2 * jnp.minimum(e, 0.0)
            ex = jnp.exp(lg - bs)
            gidx = base + i * L + lanes
            ex = jnp.where(gidx < E, ex, 0.0)
            exv[pl.ds(i * L, L)] = ex
        cp.wait()

        def _scale(i, cr):
            ex16 = exv[pl.ds(i * L, L)]
            for j in range(L):
                s = ex16[j]
                r = i * L + j
                for c8 in range(D // L):
                    rows[r, pl.ds(c8 * L, L)] = rows[r, pl.ds(c8 * L, L)] * s
            return cr

        lax.fori_loop(0, CH // L, _scale, 0)
        pltpu.sync_copy(rows, spU.at[srcv], add=True)
        pltpu.sync_copy(exv, spd.at[srcv], add=True)
        return carry

    lax.fori_loop(0, NCH, _chunk, 0)

    plsc.subcore_barrier()
    r0 = sid * ROWS_PT
    pltpu.sync_copy(spU.at[pl.ds(r0, ROWS_PT)],
                    u_ref.at[cid, pl.ds(r0, ROWS_PT)])
    pltpu.sync_copy(spd.at[pl.ds(r0, ROWS_PT)],
                    d_ref.at[cid, pl.ds(r0, ROWS_PT)])


def _make_sc():
    mesh = plsc.VectorSubcoreMesh(core_axis_name="c", subcore_axis_name="s",
                                  num_cores=NC, num_subcores=NS)
    return pl.kernel(
        _sc_body,
        out_type=[
            jax.ShapeDtypeStruct((NC, NPAD, D), jnp.float32),
            jax.ShapeDtypeStruct((NC, NPAD), jnp.float32),
        ],
        mesh=mesh,
        compiler_params=pltpu.CompilerParams(needs_layout_passes=False),
        scratch_types=[
            pltpu.VMEM((N,), jnp.float32),
            pltpu.VMEM((N,), jnp.float32),
            pltpu.VMEM((L,), jnp.float32),
            pltpu.VMEM((CH,), jnp.int32),
            pltpu.VMEM((CH,), jnp.int32),
            pltpu.VMEM((CH,), jnp.float32),
            pltpu.VMEM((CH,), jnp.float32),
            pltpu.VMEM((CH, D), jnp.float32),
            pltpu.VMEM_SHARED((NPAD, D), jnp.float32),
            pltpu.VMEM_SHARED((NPAD,), jnp.float32),
            pltpu.SemaphoreType.DMA,
        ],
    )


def _comb_body(u_ref, d_ref, b_ref, o_ref):
    us = u_ref[0] + u_ref[1]
    dns = d_ref[0] + d_ref[1]
    o_ref[...] = us / jnp.maximum(dns, 1e-16)[:, None] + b_ref[...]


_combine = pl.pallas_call(
    _comb_body,
    grid=(GC,),
    in_specs=[
        pl.BlockSpec((NC, CB, D), lambda i: (0, i, 0)),
        pl.BlockSpec((NC, CB), lambda i: (0, i)),
        pl.BlockSpec((1, D), lambda i: (0, 0)),
    ],
    out_specs=pl.BlockSpec((CB, D), lambda i: (i, 0)),
    out_shape=jax.ShapeDtypeStruct((NPAD, D), jnp.float32),
)


def kernel(x, edge_index, edge_values, W_map, a1, b1, a2, b2, kernel, bias):
    a12 = jnp.concatenate([a1, a2], axis=1)
    ev2 = edge_values.reshape(GP, 8, EB // 8)
    value, sa12, mx = _prep(x, W_map, a12, kernel, ev2)
    sa12 = sa12 + jnp.concatenate([b1, b2])[None, :]
    shift = (jnp.max(mx[:, 0]) + jnp.max(mx[:, 1])) * jnp.max(mx[:, 2])
    bsv = jnp.full((L,), shift, jnp.float32)

    pad = EPAD - E
    src = jnp.concatenate(
        [edge_index[0], jnp.zeros((pad,), jnp.int32)]).reshape(NW, NCH, CH)
    dst = jnp.concatenate(
        [edge_index[1], jnp.zeros((pad,), jnp.int32)]).reshape(NW, NCH, CH)
    evp = jnp.concatenate(
        [edge_values, jnp.zeros((pad,), jnp.float32)]).reshape(NW, NCH, CH)

    sc_fn = _make_sc()
    U, dn = sc_fn(src, dst, evp, sa12[:, 0], sa12[:, 1], value, bsv)
    out = _combine(U, dn, bias.reshape(1, D))
    return out[:N]
